# baseline (speedup 1.0000x reference)
import functools

import jax
import jax.numpy as jnp
from jax import lax
from jax.experimental import pallas as pl
from jax.experimental.pallas import tpu as pltpu
from jax.experimental.pallas import tpu_sc as plsc

N = 100000
E = 1600000
D = 32
HALF = 16
NEG_SLOPE = 0.2
EPS = 1e-9

BN = 7168
NPAD = 100352
ROWS_PER_TILE = NPAD // 16
NBLOCKS = NPAD // BN

K = 512
KROWS = K // 128
RTOT = E // 128
QTOT = RTOT // KROWS
JMAX = (QTOT + 15) // 16


_ATT_DN = (((1,), (1,)), ((), ()))


def _tc_front_body(h_ref, w_ref, alr_ref, zlo_ref, zhi_ref, att_ref):
    z = jnp.dot(h_ref[...], w_ref[...], preferred_element_type=jnp.float32)
    zlo_ref[...] = z[:, :HALF]
    zhi_ref[...] = z[:, HALF:]
    att_ref[...] = lax.dot_general(alr_ref[...], z, _ATT_DN, precision=lax.Precision.HIGHEST,
                                   preferred_element_type=jnp.float32)


def _tc_mid_body(alo_ref, ahi_ref, den_ref, blo_ref, bhi_ref,
                 wa_ref, wb_ref, alr_ref, zlo_ref, zhi_ref, att_ref):
    deni = 1.0 / (den_ref[...] + EPS)
    hlo = alo_ref[...] * deni + blo_ref[...]
    hhi = ahi_ref[...] * deni + bhi_ref[...]
    z = (jnp.dot(hlo, wa_ref[...], preferred_element_type=jnp.float32)
         + jnp.dot(hhi, wb_ref[...], preferred_element_type=jnp.float32))
    zlo_ref[...] = z[:, :HALF]
    zhi_ref[...] = z[:, HALF:]
    att_ref[...] = lax.dot_general(alr_ref[...], z, _ATT_DN, precision=lax.Precision.HIGHEST,
                                   preferred_element_type=jnp.float32)


def _tc_out_body(alo_ref, ahi_ref, den_ref, blo_ref, bhi_ref, out_ref):
    deni = 1.0 / (den_ref[...] + EPS)
    out_ref[...] = jnp.concatenate(
        [alo_ref[...] * deni + blo_ref[...],
         ahi_ref[...] * deni + bhi_ref[...]], axis=1)


_row_spec = pl.BlockSpec((BN, D), lambda i: (i, 0))
_half_spec = pl.BlockSpec((BN, HALF), lambda i: (i, 0))
_col_spec = pl.BlockSpec((BN, 1), lambda i: (i, 0))
_w_spec = pl.BlockSpec((D, D), lambda i: (0, 0))
_wh_spec = pl.BlockSpec((HALF, D), lambda i: (0, 0))
_vech_spec = pl.BlockSpec((1, HALF), lambda i: (0, 0))
_alr_spec = pl.BlockSpec((2, D), lambda i: (0, 0))
_att_spec = pl.BlockSpec((2, BN), lambda i: (0, i))

_zel_out = [
    jax.ShapeDtypeStruct((NPAD, HALF), jnp.float32),
    jax.ShapeDtypeStruct((NPAD, HALF), jnp.float32),
    jax.ShapeDtypeStruct((2, NPAD), jnp.float32),
]

_tc_front = pl.pallas_call(
    _tc_front_body,
    grid=(NBLOCKS,),
    in_specs=[_row_spec, _w_spec, _alr_spec],
    out_specs=[_half_spec, _half_spec, _att_spec],
    out_shape=_zel_out,
)

_tc_mid = pl.pallas_call(
    _tc_mid_body,
    grid=(NBLOCKS,),
    in_specs=[_half_spec, _half_spec, _col_spec, _vech_spec, _vech_spec,
              _wh_spec, _wh_spec, _alr_spec],
    out_specs=[_half_spec, _half_spec, _att_spec],
    out_shape=_zel_out,
)

_tc_out = pl.pallas_call(
    _tc_out_body,
    grid=(NBLOCKS,),
    in_specs=[_half_spec, _half_spec, _col_spec, _vech_spec, _vech_spec],
    out_specs=_row_spec,
    out_shape=jax.ShapeDtypeStruct((NPAD, D), jnp.float32),
)


@functools.cache
def _make_sc_edge_pass():
  mesh = plsc.VectorSubcoreMesh(core_axis_name="c", subcore_axis_name="s",
                                num_cores=2, num_subcores=16)

  @functools.partial(
      pl.kernel,
      mesh=mesh,
      compiler_params=pltpu.CompilerParams(use_tc_tiling_on_sc=False),
      out_type=[
          jax.ShapeDtypeStruct((NPAD, HALF), jnp.float32),
          jax.ShapeDtypeStruct((NPAD, HALF), jnp.float32),
          jax.ShapeDtypeStruct((NPAD,), jnp.float32),
      ],
      scratch_types=[
          pltpu.VMEM((2, KROWS, 128), jnp.int32),
          pltpu.VMEM((2, KROWS, 128), jnp.int32),
          pltpu.VMEM((K,), jnp.float32),
          pltpu.VMEM((K,), jnp.float32),
          pltpu.VMEM((K, HALF), jnp.float32),
          pltpu.VMEM_SHARED((NPAD,), jnp.float32),
          pltpu.VMEM_SHARED((NPAD,), jnp.float32),
          pltpu.VMEM_SHARED((NPAD,), jnp.float32),
          pltpu.VMEM_SHARED((NPAD, HALF), jnp.float32),
          pltpu.SemaphoreType.DMA,
          pltpu.SemaphoreType.DMA,
          pltpu.SemaphoreType.DMA,
          pltpu.SemaphoreType.DMA,
      ],
  )
  def _sc_edge_pass(src_hbm, dst_hbm, zlo_hbm, zhi_hbm, el_hbm, er_hbm,
                    acclo_out, acchi_out, den_out,
                    e_a, e_b, exv, erv, zrows,
                    el_sh, er_sh, den_sh, acc_sh,
                    sem_s, sem_z, sem_ia, sem_ib):
    c = lax.axis_index("c")
    s = lax.axis_index("s")
    row0 = s * ROWS_PER_TILE

    pltpu.sync_copy(el_hbm.at[pl.ds(row0, ROWS_PER_TILE)],
                    el_sh.at[pl.ds(row0, ROWS_PER_TILE)])
    pltpu.sync_copy(er_hbm.at[pl.ds(row0, ROWS_PER_TILE)],
                    er_sh.at[pl.ds(row0, ROWS_PER_TILE)])

    def _z1(i, carry):
        exv[pl.ds(i * 16, 16)] = jnp.zeros((16,), jnp.float32)
        return carry
    lax.fori_loop(0, K // 16, _z1, 0)
    for t in range(ROWS_PER_TILE // K):
        pltpu.sync_copy(exv, den_sh.at[pl.ds(row0 + t * K, K)])
    _remd = ROWS_PER_TILE % K
    if _remd:
        pltpu.sync_copy(exv.at[pl.ds(0, _remd)],
                        den_sh.at[pl.ds(row0 + ROWS_PER_TILE - _remd, _remd)])

    def _z2(i, carry):
        zrows[i] = jnp.zeros((HALF,), jnp.float32)
        return carry
    lax.fori_loop(0, K, _z2, 0)
    for t in range(ROWS_PER_TILE // K):
        pltpu.sync_copy(zrows, acc_sh.at[pl.ds(row0 + t * K, K)])
    _rem = ROWS_PER_TILE % K
    if _rem:
        pltpu.sync_copy(zrows.at[pl.ds(0, _rem)],
                        acc_sh.at[pl.ds(row0 + ROWS_PER_TILE - _rem, _rem)])

    pltpu.async_copy(src_hbm.at[pl.ds(s * KROWS, KROWS)], e_a.at[0], sem_ia)
    pltpu.async_copy(dst_hbm.at[pl.ds(s * KROWS, KROWS)], e_a.at[1], sem_ia)
    pltpu.async_copy(src_hbm.at[pl.ds((s + 16) * KROWS, KROWS)],
                     e_b.at[0], sem_ib)
    pltpu.async_copy(dst_hbm.at[pl.ds((s + 16) * KROWS, KROWS)],
                     e_b.at[1], sem_ib)

    plsc.subcore_barrier()

    def process(j, ebuf, sem_i):
        q = s + 16 * j

        @pl.when(q < QTOT)
        def _():
            pltpu.make_async_copy(src_hbm.at[pl.ds(0, KROWS)],
                                  ebuf.at[0], sem_i).wait()
            pltpu.make_async_copy(dst_hbm.at[pl.ds(0, KROWS)],
                                  ebuf.at[1], sem_i).wait()

            @pl.when(c == 0)
            def _():
                for i in range(KROWS):
                    pltpu.async_copy(zlo_hbm.at[ebuf.at[0, i]],
                                     zrows.at[pl.ds(i * 128, 128)], sem_z)

            @pl.when(c != 0)
            def _():
                for i in range(KROWS):
                    pltpu.async_copy(zhi_hbm.at[ebuf.at[0, i]],
                                     zrows.at[pl.ds(i * 128, 128)], sem_z)

            scps = []
            for i in range(KROWS):
                scps.append(pltpu.async_copy(
                    el_sh.at[ebuf.at[0, i]], exv.at[pl.ds(i * 128, 128)],
                    sem_s))
                scps.append(pltpu.async_copy(
                    er_sh.at[ebuf.at[1, i]], erv.at[pl.ds(i * 128, 128)],
                    sem_s))
            for cp in scps:
                cp.wait()

            def ex_body(i, carry):
                e = exv[pl.ds(i * 16, 16)] + erv[pl.ds(i * 16, 16)]
                e = jnp.where(e >= 0.0, e, e * NEG_SLOPE)
                exv[pl.ds(i * 16, 16)] = jnp.exp(e)
                return carry
            lax.fori_loop(0, K // 16, ex_body, 0)

            @pl.when(c == 0)
            def _():
                for i in range(KROWS):
                    pltpu.async_copy(exv.at[pl.ds(i * 128, 128)],
                                     den_sh.at[ebuf.at[1, i]], sem_s,
                                     add=True)

            for i in range(KROWS):
                pltpu.make_async_copy(
                    zlo_hbm.at[ebuf.at[0, i]],
                    zrows.at[pl.ds(i * 128, 128)], sem_z).wait()

            def mul_body(i, carry):
                ex16 = exv[pl.ds(i * 16, 16)]
                for l in range(16):
                    exb = jnp.broadcast_to(ex16[l], (16,))
                    zrows[i * 16 + l] = zrows[i * 16 + l] * exb
                return carry
            lax.fori_loop(0, K // 16, mul_body, 0)

            acps = []
            for i in range(KROWS):
                acps.append(pltpu.async_copy(
                    zrows.at[pl.ds(i * 128, 128)],
                    acc_sh.at[ebuf.at[1, i]], sem_z, add=True))
            @pl.when(c == 0)
            def _():
                for i in range(KROWS):
                    pltpu.make_async_copy(
                        exv.at[pl.ds(i * 128, 128)],
                        den_sh.at[ebuf.at[1, i]], sem_s).wait()
            for cp in acps:
                cp.wait()

            @pl.when(q + 32 < QTOT)
            def _():
                rn = (q + 32) * KROWS
                pltpu.async_copy(src_hbm.at[pl.ds(rn, KROWS)],
                                 ebuf.at[0], sem_i)
                pltpu.async_copy(dst_hbm.at[pl.ds(rn, KROWS)],
                                 ebuf.at[1], sem_i)

    def pair_body(m, carry):
        process(2 * m, e_a, sem_ia)
        process(2 * m + 1, e_b, sem_ib)
        return carry

    lax.fori_loop(0, JMAX // 2, pair_body, 0)

    plsc.subcore_barrier()

    @pl.when(c == 0)
    def _():
        pltpu.sync_copy(acc_sh.at[pl.ds(row0, ROWS_PER_TILE)],
                        acclo_out.at[pl.ds(row0, ROWS_PER_TILE)])
        pltpu.sync_copy(den_sh.at[pl.ds(row0, ROWS_PER_TILE)],
                        den_out.at[pl.ds(row0, ROWS_PER_TILE)])

    @pl.when(c != 0)
    def _():
        pltpu.sync_copy(acc_sh.at[pl.ds(row0, ROWS_PER_TILE)],
                        acchi_out.at[pl.ds(row0, ROWS_PER_TILE)])

  return _sc_edge_pass


def kernel(h_inputs, edge_index, objectives, W0, al0, ar0, b0,
           W1, al1, ar1, b1):
    h = jnp.concatenate([h_inputs, objectives], axis=1)
    h = jnp.pad(h, ((0, NPAD - N), (0, 0)))

    src2d = edge_index[0].reshape(RTOT, 128)
    dst2d = edge_index[1].reshape(RTOT, 128)

    sc_edge_pass = _make_sc_edge_pass()

    alr0 = jnp.concatenate([al0, ar0], axis=0)
    alr1 = jnp.concatenate([al1, ar1], axis=0)
    b0lo, b0hi = b0[:HALF].reshape(1, HALF), b0[HALF:].reshape(1, HALF)
    b1lo, b1hi = b1[:HALF].reshape(1, HALF), b1[HALF:].reshape(1, HALF)
    W1a, W1b = W1[:HALF, :], W1[HALF:, :]

    zlo, zhi, att = _tc_front(h, W0, alr0)
    alo, ahi, den = sc_edge_pass(src2d, dst2d, zlo, zhi, att[0], att[1])

    zlo2, zhi2, att2 = _tc_mid(alo, ahi, den.reshape(NPAD, 1),
                               b0lo, b0hi, W1a, W1b, alr1)
    alo2, ahi2, den2 = sc_edge_pass(src2d, dst2d, zlo2, zhi2,
                                    att2[0], att2[1])

    out = _tc_out(alo2, ahi2, den2.reshape(NPAD, 1), b1lo, b1hi)
    return out[:N]


# speedup vs baseline: 1.0023x; 1.0023x over previous
import functools

import jax
import jax.numpy as jnp
from jax import lax
from jax.experimental import pallas as pl
from jax.experimental.pallas import tpu as pltpu
from jax.experimental.pallas import tpu_sc as plsc

N = 100000
E = 1600000
D = 32
HALF = 16
NEG_SLOPE = 0.2
EPS = 1e-9

BN = 7168
NPAD = 100352
ROWS_PER_TILE = NPAD // 16
NBLOCKS = NPAD // BN

K = 512
KROWS = K // 128
RTOT = E // 128
QTOT = RTOT // KROWS
JMAX = (QTOT + 15) // 16


_ATT_DN = (((1,), (1,)), ((), ()))


def _tc_front_body(h_ref, w_ref, alr_ref, zlo_ref, zhi_ref, att_ref):
    z = jnp.dot(h_ref[...], w_ref[...], preferred_element_type=jnp.float32)
    zlo_ref[...] = z[:, :HALF]
    zhi_ref[...] = z[:, HALF:]
    att_ref[...] = lax.dot_general(alr_ref[...], z, _ATT_DN, precision=lax.Precision.HIGHEST,
                                   preferred_element_type=jnp.float32)


def _tc_mid_body(alo_ref, ahi_ref, den_ref, blo_ref, bhi_ref,
                 wa_ref, wb_ref, alr_ref, zlo_ref, zhi_ref, att_ref):
    deni = 1.0 / (den_ref[...] + EPS)
    hlo = alo_ref[...] * deni + blo_ref[...]
    hhi = ahi_ref[...] * deni + bhi_ref[...]
    z = (jnp.dot(hlo, wa_ref[...], preferred_element_type=jnp.float32)
         + jnp.dot(hhi, wb_ref[...], preferred_element_type=jnp.float32))
    zlo_ref[...] = z[:, :HALF]
    zhi_ref[...] = z[:, HALF:]
    att_ref[...] = lax.dot_general(alr_ref[...], z, _ATT_DN, precision=lax.Precision.HIGHEST,
                                   preferred_element_type=jnp.float32)


def _tc_out_body(alo_ref, ahi_ref, den_ref, blo_ref, bhi_ref, out_ref):
    deni = 1.0 / (den_ref[...] + EPS)
    out_ref[...] = jnp.concatenate(
        [alo_ref[...] * deni + blo_ref[...],
         ahi_ref[...] * deni + bhi_ref[...]], axis=1)


_row_spec = pl.BlockSpec((BN, D), lambda i: (i, 0))
_half_spec = pl.BlockSpec((BN, HALF), lambda i: (i, 0))
_col_spec = pl.BlockSpec((BN, 1), lambda i: (i, 0))
_w_spec = pl.BlockSpec((D, D), lambda i: (0, 0))
_wh_spec = pl.BlockSpec((HALF, D), lambda i: (0, 0))
_vech_spec = pl.BlockSpec((1, HALF), lambda i: (0, 0))
_alr_spec = pl.BlockSpec((2, D), lambda i: (0, 0))
_att_spec = pl.BlockSpec((2, BN), lambda i: (0, i))

_zel_out = [
    jax.ShapeDtypeStruct((NPAD, HALF), jnp.float32),
    jax.ShapeDtypeStruct((NPAD, HALF), jnp.float32),
    jax.ShapeDtypeStruct((2, NPAD), jnp.float32),
]

_tc_front = pl.pallas_call(
    _tc_front_body,
    grid=(NBLOCKS,),
    in_specs=[_row_spec, _w_spec, _alr_spec],
    out_specs=[_half_spec, _half_spec, _att_spec],
    out_shape=_zel_out,
)

_tc_mid = pl.pallas_call(
    _tc_mid_body,
    grid=(NBLOCKS,),
    in_specs=[_half_spec, _half_spec, _col_spec, _vech_spec, _vech_spec,
              _wh_spec, _wh_spec, _alr_spec],
    out_specs=[_half_spec, _half_spec, _att_spec],
    out_shape=_zel_out,
)

_tc_out = pl.pallas_call(
    _tc_out_body,
    grid=(NBLOCKS,),
    in_specs=[_half_spec, _half_spec, _col_spec, _vech_spec, _vech_spec],
    out_specs=_row_spec,
    out_shape=jax.ShapeDtypeStruct((NPAD, D), jnp.float32),
)


@functools.cache
def _make_sc_edge_pass():
  mesh = plsc.VectorSubcoreMesh(core_axis_name="c", subcore_axis_name="s",
                                num_cores=2, num_subcores=16)

  @functools.partial(
      pl.kernel,
      mesh=mesh,
      compiler_params=pltpu.CompilerParams(use_tc_tiling_on_sc=False),
      out_type=[
          jax.ShapeDtypeStruct((NPAD, HALF), jnp.float32),
          jax.ShapeDtypeStruct((NPAD, HALF), jnp.float32),
          jax.ShapeDtypeStruct((NPAD,), jnp.float32),
      ],
      scratch_types=[
          pltpu.VMEM((2, KROWS, 128), jnp.int32),
          pltpu.VMEM((2, KROWS, 128), jnp.int32),
          pltpu.VMEM((K,), jnp.float32),
          pltpu.VMEM((K,), jnp.float32),
          pltpu.VMEM((K, HALF), jnp.float32),
          pltpu.VMEM_SHARED((NPAD,), jnp.float32),
          pltpu.VMEM_SHARED((NPAD,), jnp.float32),
          pltpu.VMEM_SHARED((NPAD,), jnp.float32),
          pltpu.VMEM_SHARED((NPAD, HALF), jnp.float32),
          pltpu.SemaphoreType.DMA,
          pltpu.SemaphoreType.DMA,
          pltpu.SemaphoreType.DMA,
          pltpu.SemaphoreType.DMA,
      ],
  )
  def _sc_edge_pass(src_hbm, dst_hbm, zlo_hbm, zhi_hbm, el_hbm, er_hbm,
                    acclo_out, acchi_out, den_out,
                    e_a, e_b, exv, erv, zrows,
                    el_sh, er_sh, den_sh, acc_sh,
                    sem_s, sem_z, sem_ia, sem_ib):
    c = lax.axis_index("c")
    s = lax.axis_index("s")
    row0 = s * ROWS_PER_TILE

    pcps = [
        pltpu.async_copy(el_hbm.at[pl.ds(row0, ROWS_PER_TILE)],
                         el_sh.at[pl.ds(row0, ROWS_PER_TILE)], sem_s),
        pltpu.async_copy(er_hbm.at[pl.ds(row0, ROWS_PER_TILE)],
                         er_sh.at[pl.ds(row0, ROWS_PER_TILE)], sem_s),
    ]

    def _z1(i, carry):
        exv[pl.ds(i * 16, 16)] = jnp.zeros((16,), jnp.float32)
        return carry
    lax.fori_loop(0, K // 16, _z1, 0, unroll=4)

    def _z2(i, carry):
        zrows[i] = jnp.zeros((HALF,), jnp.float32)
        return carry
    lax.fori_loop(0, K, _z2, 0, unroll=4)

    for t in range(ROWS_PER_TILE // K):
        pcps.append(pltpu.async_copy(
            exv, den_sh.at[pl.ds(row0 + t * K, K)], sem_s))
        pcps.append(pltpu.async_copy(
            zrows, acc_sh.at[pl.ds(row0 + t * K, K)], sem_s))
    _remd = ROWS_PER_TILE % K
    if _remd:
        pcps.append(pltpu.async_copy(
            exv.at[pl.ds(0, _remd)],
            den_sh.at[pl.ds(row0 + ROWS_PER_TILE - _remd, _remd)], sem_s))
        pcps.append(pltpu.async_copy(
            zrows.at[pl.ds(0, _remd)],
            acc_sh.at[pl.ds(row0 + ROWS_PER_TILE - _remd, _remd)], sem_s))
    for cp in pcps:
        cp.wait()

    pltpu.async_copy(src_hbm.at[pl.ds(s * KROWS, KROWS)], e_a.at[0], sem_ia)
    pltpu.async_copy(dst_hbm.at[pl.ds(s * KROWS, KROWS)], e_a.at[1], sem_ia)
    pltpu.async_copy(src_hbm.at[pl.ds((s + 16) * KROWS, KROWS)],
                     e_b.at[0], sem_ib)
    pltpu.async_copy(dst_hbm.at[pl.ds((s + 16) * KROWS, KROWS)],
                     e_b.at[1], sem_ib)

    plsc.subcore_barrier()

    def process(j, ebuf, sem_i):
        q = s + 16 * j

        @pl.when(q < QTOT)
        def _():
            pltpu.make_async_copy(src_hbm.at[pl.ds(0, KROWS)],
                                  ebuf.at[0], sem_i).wait()
            pltpu.make_async_copy(dst_hbm.at[pl.ds(0, KROWS)],
                                  ebuf.at[1], sem_i).wait()

            @pl.when(c == 0)
            def _():
                for i in range(KROWS):
                    pltpu.async_copy(zlo_hbm.at[ebuf.at[0, i]],
                                     zrows.at[pl.ds(i * 128, 128)], sem_z)

            @pl.when(c != 0)
            def _():
                for i in range(KROWS):
                    pltpu.async_copy(zhi_hbm.at[ebuf.at[0, i]],
                                     zrows.at[pl.ds(i * 128, 128)], sem_z)

            scps = []
            for i in range(KROWS):
                scps.append(pltpu.async_copy(
                    el_sh.at[ebuf.at[0, i]], exv.at[pl.ds(i * 128, 128)],
                    sem_s))
                scps.append(pltpu.async_copy(
                    er_sh.at[ebuf.at[1, i]], erv.at[pl.ds(i * 128, 128)],
                    sem_s))
            for cp in scps:
                cp.wait()

            def ex_body(i, carry):
                e = exv[pl.ds(i * 16, 16)] + erv[pl.ds(i * 16, 16)]
                e = jnp.where(e >= 0.0, e, e * NEG_SLOPE)
                exv[pl.ds(i * 16, 16)] = jnp.exp(e)
                return carry
            lax.fori_loop(0, K // 16, ex_body, 0, unroll=4)

            @pl.when(c == 0)
            def _():
                for i in range(KROWS):
                    pltpu.async_copy(exv.at[pl.ds(i * 128, 128)],
                                     den_sh.at[ebuf.at[1, i]], sem_s,
                                     add=True)

            for i in range(KROWS):
                pltpu.make_async_copy(
                    zlo_hbm.at[ebuf.at[0, i]],
                    zrows.at[pl.ds(i * 128, 128)], sem_z).wait()

            def mul_body(i, carry):
                ex16 = exv[pl.ds(i * 16, 16)]
                for l in range(16):
                    exb = jnp.broadcast_to(ex16[l], (16,))
                    zrows[i * 16 + l] = zrows[i * 16 + l] * exb
                return carry
            lax.fori_loop(0, K // 16, mul_body, 0, unroll=2)

            acps = []
            for i in range(KROWS):
                acps.append(pltpu.async_copy(
                    zrows.at[pl.ds(i * 128, 128)],
                    acc_sh.at[ebuf.at[1, i]], sem_z, add=True))
            @pl.when(c == 0)
            def _():
                for i in range(KROWS):
                    pltpu.make_async_copy(
                        exv.at[pl.ds(i * 128, 128)],
                        den_sh.at[ebuf.at[1, i]], sem_s).wait()
            for cp in acps:
                cp.wait()

            @pl.when(q + 32 < QTOT)
            def _():
                rn = (q + 32) * KROWS
                pltpu.async_copy(src_hbm.at[pl.ds(rn, KROWS)],
                                 ebuf.at[0], sem_i)
                pltpu.async_copy(dst_hbm.at[pl.ds(rn, KROWS)],
                                 ebuf.at[1], sem_i)

    def pair_body(m, carry):
        process(2 * m, e_a, sem_ia)
        process(2 * m + 1, e_b, sem_ib)
        return carry

    lax.fori_loop(0, JMAX // 2, pair_body, 0)

    plsc.subcore_barrier()

    @pl.when(c == 0)
    def _():
        pltpu.sync_copy(acc_sh.at[pl.ds(row0, ROWS_PER_TILE)],
                        acclo_out.at[pl.ds(row0, ROWS_PER_TILE)])
        pltpu.sync_copy(den_sh.at[pl.ds(row0, ROWS_PER_TILE)],
                        den_out.at[pl.ds(row0, ROWS_PER_TILE)])

    @pl.when(c != 0)
    def _():
        pltpu.sync_copy(acc_sh.at[pl.ds(row0, ROWS_PER_TILE)],
                        acchi_out.at[pl.ds(row0, ROWS_PER_TILE)])

  return _sc_edge_pass


def kernel(h_inputs, edge_index, objectives, W0, al0, ar0, b0,
           W1, al1, ar1, b1):
    h = jnp.concatenate([h_inputs, objectives], axis=1)
    h = jnp.pad(h, ((0, NPAD - N), (0, 0)))

    src2d = edge_index[0].reshape(RTOT, 128)
    dst2d = edge_index[1].reshape(RTOT, 128)

    sc_edge_pass = _make_sc_edge_pass()

    alr0 = jnp.concatenate([al0, ar0], axis=0)
    alr1 = jnp.concatenate([al1, ar1], axis=0)
    b0lo, b0hi = b0[:HALF].reshape(1, HALF), b0[HALF:].reshape(1, HALF)
    b1lo, b1hi = b1[:HALF].reshape(1, HALF), b1[HALF:].reshape(1, HALF)
    W1a, W1b = W1[:HALF, :], W1[HALF:, :]

    zlo, zhi, att = _tc_front(h, W0, alr0)
    alo, ahi, den = sc_edge_pass(src2d, dst2d, zlo, zhi, att[0], att[1])

    zlo2, zhi2, att2 = _tc_mid(alo, ahi, den.reshape(NPAD, 1),
                               b0lo, b0hi, W1a, W1b, alr1)
    alo2, ahi2, den2 = sc_edge_pass(src2d, dst2d, zlo2, zhi2,
                                    att2[0], att2[1])

    out = _tc_out(alo2, ahi2, den2.reshape(NPAD, 1), b1lo, b1hi)
    return out[:N]
